# baseline (speedup 1.0000x reference)
import jax
import jax.numpy as jnp
import numpy as np
from jax.experimental import pallas as pl

N = 10000
HID = 128


def _gatv2(x, src, dst, ea, Wl, bl, Wr, br, We, att, b, n_nodes):
    loop = jnp.arange(n_nodes, dtype=src.dtype)
    src2 = jnp.concatenate([src, loop])
    dst2 = jnp.concatenate([dst, loop])
    mean_ea = jnp.mean(ea, axis=0, keepdims=True)
    ea2 = jnp.concatenate([ea, jnp.broadcast_to(mean_ea, (n_nodes, ea.shape[1]))], axis=0)
    xl = x @ Wl.T + bl
    xr = x @ Wr.T + br
    xj = xl[src2]
    xi = xr[dst2]
    ef = ea2 @ We.T
    h = jax.nn.leaky_relu(xj + xi + ef, negative_slope=0.2)
    alpha = h @ att
    amax = jax.ops.segment_max(alpha, dst2, num_segments=n_nodes)
    ex = jnp.exp(alpha - amax[dst2])
    den = jax.ops.segment_sum(ex, dst2, num_segments=n_nodes)
    w = ex / (den[dst2] + 1e-16)
    return jax.ops.segment_sum(xj * w[:, None], dst2, num_segments=n_nodes) + b


def _mha(x, Win, bin_, Wout, bout, nh):
    L, Emb = x.shape
    hd = Emb // nh
    qkv = x @ Win.T + bin_
    q, k, v = jnp.split(qkv, 3, axis=-1)
    q = q.reshape(L, nh, hd).transpose(1, 0, 2)
    k = k.reshape(L, nh, hd).transpose(1, 0, 2)
    v = v.reshape(L, nh, hd).transpose(1, 0, 2)
    s = jnp.matmul(q, k.transpose(0, 2, 1)) * (1.0 / float(np.sqrt(hd)))
    a = jax.nn.softmax(s, axis=-1)
    o = jnp.matmul(a, v).transpose(1, 0, 2).reshape(L, Emb)
    return o @ Wout.T + bout


def _gru(hist, Wih, Whh, bih, bhh, hid):
    h = jnp.zeros((hid,), dtype=hist.dtype)
    outs = []
    for t in range(hist.shape[0]):
        gx = hist[t] @ Wih.T + bih
        gh = h @ Whh.T + bhh
        xr, xz, xn = jnp.split(gx, 3)
        hr, hz, hn = jnp.split(gh, 3)
        r = jax.nn.sigmoid(xr + hr)
        z = jax.nn.sigmoid(xz + hz)
        n = jnp.tanh(xn + r * hn)
        h = (1.0 - z) * n + z * h
        outs.append(h)
    return jnp.stack(outs, axis=0)


def _identity_kernel(x_ref, o_ref):
    o_ref[...] = x_ref[...]


def kernel(x, edge_index, edge_attr, history, c1_Wl, c1_bl, c1_Wr, c1_br, c1_We, c1_att, c1_b, c2_Wl, c2_bl, c2_Wr, c2_br, c2_We, c2_att, c2_b, p_Win, p_bin, p_Wout, p_bout, m_W1, m_b1, m_W2, m_b2, g_Wih, g_Whh, g_bih, g_bhh, h_W1, h_b1, h_W2, h_b2):
    src = edge_index[0]
    dst = edge_index[1]
    h1 = jax.nn.relu(_gatv2(x, src, dst, edge_attr, c1_Wl, c1_bl, c1_Wr, c1_br, c1_We, c1_att, c1_b, N))
    h2 = _gatv2(h1, src, dst, edge_attr, c2_Wl, c2_bl, c2_Wr, c2_br, c2_We, c2_att, c2_b, N)
    a = _mha(h2, p_Win, p_bin, p_Wout, p_bout, 2)
    a = jnp.max(a, axis=0)
    m = jax.nn.leaky_relu(a.reshape(1, -1) @ m_W1.T + m_b1, 0.01)
    m = jax.nn.leaky_relu(m @ m_W2.T + m_b2, 0.01)
    g = _gru(history, g_Wih, g_Whh, g_bih, g_bhh, HID).reshape(1, -1)
    cat = jnp.concatenate([m, g], axis=1)
    o = jax.nn.leaky_relu(cat @ h_W1.T + h_b1, 0.01)
    o = pl.pallas_call(
        _identity_kernel,
        out_shape=jax.ShapeDtypeStruct(o.shape, o.dtype),
    )(o)
    return o @ h_W2.T + h_b2


# speedup vs baseline: 4.4035x; 4.4035x over previous
import functools

import jax
import jax.numpy as jnp
import numpy as np
from jax import lax
from jax.experimental import pallas as pl
from jax.experimental.pallas import tpu as pltpu
from jax.experimental.pallas import tpu_sc as plsc

N = 10000
E = 320000
HID = 128
EDIM = 4
ROWW = 128

NC, NS = 2, 16
NW = NC * NS
EPT = E // NW
CH = 80
NCHUNK = EPT // CH
NBLK = 10
BN = N // NBLK
EBLK = 125
BE = E // EBLK
STR = 624
STRL = N - 15 * STR

NEG = -1e30


def _eamean_body(ea_ref, o_ref):
    s = jnp.sum(ea_ref[...], axis=0)
    o_ref[...] = jnp.broadcast_to(s[None, :], (8, 128))


def _ea_colsum(ea_flat2d):
    return pl.pallas_call(
        _eamean_body,
        out_shape=jax.ShapeDtypeStruct((8, 128), jnp.float32),
    )(ea_flat2d)


def _pre_body(x_ref, wl_ref, bl_ref, wr_ref, br_ref, efm_ref, att_ref,
              xl_ref, xr_ref, as_ref, mx_ref):
    i = pl.program_id(0)
    x = x_ref[...]
    xl = jax.lax.dot_general(x, wl_ref[...], (((1,), (1,)), ((), ())),
                             preferred_element_type=jnp.float32) + bl_ref[...]
    xr = jax.lax.dot_general(x, wr_ref[...], (((1,), (1,)), ((), ())),
                             preferred_element_type=jnp.float32) + br_ref[...]
    xl_ref[...] = xl
    xr_ref[...] = xr
    u = xl + xr + efm_ref[...]
    u = jnp.where(u > 0, u, 0.2 * u)
    a_self = jnp.sum(u * att_ref[...], axis=1, keepdims=True)
    as_ref[...] = a_self

    @pl.when(i == 0)
    def _():
        mx_ref[...] = jnp.full((8, 128), NEG, jnp.float32)

    mx_ref[...] = jnp.maximum(mx_ref[...], jnp.max(a_self))


def _pre(x, Wl, bl, Wr, br, efm, att):
    return pl.pallas_call(
        _pre_body,
        grid=(NBLK,),
        in_specs=[
            pl.BlockSpec((BN, HID), lambda i: (i, 0)),
            pl.BlockSpec((HID, HID), lambda i: (0, 0)),
            pl.BlockSpec((1, HID), lambda i: (0, 0)),
            pl.BlockSpec((HID, HID), lambda i: (0, 0)),
            pl.BlockSpec((1, HID), lambda i: (0, 0)),
            pl.BlockSpec((1, HID), lambda i: (0, 0)),
            pl.BlockSpec((1, HID), lambda i: (0, 0)),
        ],
        out_specs=[
            pl.BlockSpec((BN, HID), lambda i: (i, 0)),
            pl.BlockSpec((BN, HID), lambda i: (i, 0)),
            pl.BlockSpec((BN, 1), lambda i: (i, 0)),
            pl.BlockSpec((8, 128), lambda i: (0, 0)),
        ],
        out_shape=[
            jax.ShapeDtypeStruct((N, HID), jnp.float32),
            jax.ShapeDtypeStruct((N, HID), jnp.float32),
            jax.ShapeDtypeStruct((N, 1), jnp.float32),
            jax.ShapeDtypeStruct((8, 128), jnp.float32),
        ],
    )(x, Wl, bl, Wr, br, efm, att)


def _alpha_body(xj_ref, xi_ref, ea_ref, wet_ref, att_ref, al_ref, mx_ref):
    i = pl.program_id(0)
    ef = jax.lax.dot_general(ea_ref[...], wet_ref[...], (((1,), (0,)), ((), ())),
                             preferred_element_type=jnp.float32)
    u = xj_ref[...] + xi_ref[...] + ef
    u = jnp.where(u > 0, u, 0.2 * u)
    a = jnp.sum(u * att_ref[...], axis=1, keepdims=True)
    al_ref[...] = a

    @pl.when(i == 0)
    def _():
        mx_ref[...] = jnp.full((8, 128), NEG, jnp.float32)

    mx_ref[...] = jnp.maximum(mx_ref[...], jnp.max(a))


def _alpha(xj, xi, ea, wet, att):
    return pl.pallas_call(
        _alpha_body,
        grid=(EBLK,),
        in_specs=[
            pl.BlockSpec((BE, HID), lambda i: (i, 0)),
            pl.BlockSpec((BE, HID), lambda i: (i, 0)),
            pl.BlockSpec((BE, EDIM), lambda i: (i, 0)),
            pl.BlockSpec((EDIM, HID), lambda i: (0, 0)),
            pl.BlockSpec((1, HID), lambda i: (0, 0)),
        ],
        out_specs=[
            pl.BlockSpec((BE, 1), lambda i: (i, 0)),
            pl.BlockSpec((8, 128), lambda i: (0, 0)),
        ],
        out_shape=[
            jax.ShapeDtypeStruct((E, 1), jnp.float32),
            jax.ShapeDtypeStruct((8, 128), jnp.float32),
        ],
    )(xj, xi, ea, wet, att)


def _exscale_body(xj_ref, al_ref, m1_ref, m2_ref, o_ref, ex_ref):
    m = jnp.maximum(jnp.max(m1_ref[...]), jnp.max(m2_ref[...]))
    ex = jnp.exp(al_ref[...] - m)
    o_ref[...] = xj_ref[...] * ex
    ex_ref[...] = ex


def _exscale(xj, alpha, m1, m2):
    return pl.pallas_call(
        _exscale_body,
        grid=(EBLK,),
        in_specs=[
            pl.BlockSpec((BE, HID), lambda i: (i, 0)),
            pl.BlockSpec((BE, 1), lambda i: (i, 0)),
            pl.BlockSpec((8, 128), lambda i: (0, 0)),
            pl.BlockSpec((8, 128), lambda i: (0, 0)),
        ],
        out_specs=[
            pl.BlockSpec((BE, ROWW), lambda i: (i, 0)),
            pl.BlockSpec((BE, 1), lambda i: (i, 0)),
        ],
        out_shape=[
            jax.ShapeDtypeStruct((E, ROWW), jnp.float32),
            jax.ShapeDtypeStruct((E, 1), jnp.float32),
        ],
    )(xj, alpha, m1, m2)


def _comb_body(nums_ref, dens_ref, xl_ref, xr_ref, as_ref, m1_ref, m2_ref,
               b_ref, wl2_ref, bl2_ref, wr2_ref, br2_ref, efm2_ref, att2_ref,
               relu_ref,
               xl2_ref, xr2_ref, as2_ref, mx2_ref):
    i = pl.program_id(0)
    m = jnp.maximum(jnp.max(m1_ref[...]), jnp.max(m2_ref[...]))
    exs = jnp.exp(as_ref[...] - m)
    xl = xl_ref[...]
    den = jnp.sum(dens_ref[...], axis=1, keepdims=True) + exs + 1e-16
    num = nums_ref[0] + nums_ref[1] + exs * xl
    h = num / den + b_ref[...]
    h = jnp.where(relu_ref[0, 0] > 0, jnp.maximum(h, 0.0), h)
    xl2 = jax.lax.dot_general(h, wl2_ref[...], (((1,), (1,)), ((), ())),
                              preferred_element_type=jnp.float32) + bl2_ref[...]
    xr2 = jax.lax.dot_general(h, wr2_ref[...], (((1,), (1,)), ((), ())),
                              preferred_element_type=jnp.float32) + br2_ref[...]
    xl2_ref[...] = xl2
    xr2_ref[...] = xr2
    u = xl2 + xr2 + efm2_ref[...]
    u = jnp.where(u > 0, u, 0.2 * u)
    a2 = jnp.sum(u * att2_ref[...], axis=1, keepdims=True)
    as2_ref[...] = a2

    @pl.when(i == 0)
    def _():
        mx2_ref[...] = jnp.full((8, 128), NEG, jnp.float32)

    mx2_ref[...] = jnp.maximum(mx2_ref[...], jnp.max(a2))


def _combine(nums, dens, xl, xr, a_self, m1, m2, b, Wl2, bl2, Wr2, br2, efm2,
             att2, relu_flag):
    return pl.pallas_call(
        _comb_body,
        grid=(NBLK,),
        in_specs=[
            pl.BlockSpec((2, BN, ROWW), lambda i: (0, i, 0)),
            pl.BlockSpec((BN, NC), lambda i: (i, 0)),
            pl.BlockSpec((BN, HID), lambda i: (i, 0)),
            pl.BlockSpec((BN, HID), lambda i: (i, 0)),
            pl.BlockSpec((BN, 1), lambda i: (i, 0)),
            pl.BlockSpec((8, 128), lambda i: (0, 0)),
            pl.BlockSpec((8, 128), lambda i: (0, 0)),
            pl.BlockSpec((1, HID), lambda i: (0, 0)),
            pl.BlockSpec((HID, HID), lambda i: (0, 0)),
            pl.BlockSpec((1, HID), lambda i: (0, 0)),
            pl.BlockSpec((HID, HID), lambda i: (0, 0)),
            pl.BlockSpec((1, HID), lambda i: (0, 0)),
            pl.BlockSpec((1, HID), lambda i: (0, 0)),
            pl.BlockSpec((1, HID), lambda i: (0, 0)),
            pl.BlockSpec((1, 1), lambda i: (0, 0), memory_space=pltpu.SMEM),
        ],
        out_specs=[
            pl.BlockSpec((BN, HID), lambda i: (i, 0)),
            pl.BlockSpec((BN, HID), lambda i: (i, 0)),
            pl.BlockSpec((BN, 1), lambda i: (i, 0)),
            pl.BlockSpec((8, 128), lambda i: (0, 0)),
        ],
        out_shape=[
            jax.ShapeDtypeStruct((N, HID), jnp.float32),
            jax.ShapeDtypeStruct((N, HID), jnp.float32),
            jax.ShapeDtypeStruct((N, 1), jnp.float32),
            jax.ShapeDtypeStruct((8, 128), jnp.float32),
        ],
    )(nums, dens, xl, xr, a_self, m1, m2, b, Wl2, bl2, Wr2, br2, efm2, att2,
      relu_flag)


def _gather_body(xl_hbm, xr_hbm, src_hbm, dst_hbm, xj_hbm, xi_hbm,
                 sidx, didx, bj, bi, sem):
    cid = lax.axis_index("c")
    sid = lax.axis_index("s")
    wid = sid * NC + cid
    base0 = wid * EPT

    def step(c, _):
        base = base0 + c * CH
        pltpu.sync_copy(src_hbm.at[pl.ds(base, CH)], sidx.at[0])
        pltpu.sync_copy(dst_hbm.at[pl.ds(base, CH)], didx.at[0])
        cj = pltpu.async_copy(xl_hbm.at[sidx.at[0]], bj, sem)
        ci = pltpu.async_copy(xr_hbm.at[didx.at[0]], bi, sem)
        cj.wait()
        ci.wait()
        pltpu.sync_copy(bj, xj_hbm.at[pl.ds(base, CH)])
        pltpu.sync_copy(bi, xi_hbm.at[pl.ds(base, CH)])
        return _

    lax.fori_loop(0, NCHUNK, step, 0)


def _gather(xl, xr, src, dst):
    mesh = plsc.VectorSubcoreMesh(core_axis_name="c", subcore_axis_name="s",
                                  num_cores=NC, num_subcores=NS)
    f = pl.kernel(
        _gather_body,
        out_type=[
            jax.ShapeDtypeStruct((E, HID), jnp.float32),
            jax.ShapeDtypeStruct((E, HID), jnp.float32),
        ],
        mesh=mesh,
        scratch_types=[
            pltpu.VMEM((1, CH), jnp.int32),
            pltpu.VMEM((1, CH), jnp.int32),
            pltpu.VMEM((CH, HID), jnp.float32),
            pltpu.VMEM((CH, HID), jnp.float32),
            pltpu.SemaphoreType.DMA,
        ],
    )
    return f(xl, xr, src, dst)


def _scatter_body(xjs_hbm, ex_hbm, dst_hbm, zeros_hbm, zeros1_hbm,
                  out_hbm, dens_hbm, acc, accden, didx, rows, exbuf, zbuf):
    cid = lax.axis_index("c")
    sid = lax.axis_index("s")
    base0 = (sid * NC + cid) * EPT

    pltpu.sync_copy(zeros1_hbm.at[pl.ds(0, STRL)], zbuf)

    @pl.when(sid < 15)
    def _():
        pltpu.sync_copy(zeros_hbm.at[pl.ds(sid * STR, STR)],
                        acc.at[pl.ds(sid * STR, STR)])
        pltpu.sync_copy(zbuf.at[pl.ds(0, STR)],
                        accden.at[pl.ds(sid * STR, STR)])

    @pl.when(sid == 15)
    def _():
        pltpu.sync_copy(zeros_hbm.at[pl.ds(15 * STR, STRL)],
                        acc.at[pl.ds(15 * STR, STRL)])
        pltpu.sync_copy(zbuf,
                        accden.at[pl.ds(15 * STR, STRL)])

    plsc.subcore_barrier()

    def step(c, _):
        base = base0 + c * CH
        pltpu.sync_copy(dst_hbm.at[pl.ds(base, CH)], didx.at[0])
        pltpu.sync_copy(xjs_hbm.at[pl.ds(base, CH)], rows)
        pltpu.sync_copy(ex_hbm.at[pl.ds(base, CH)], exbuf)
        pltpu.sync_copy(rows, acc.at[didx.at[0]], add=True)
        pltpu.sync_copy(exbuf, accden.at[didx.at[0]], add=True)
        return _

    lax.fori_loop(0, NCHUNK, step, 0)
    plsc.subcore_barrier()

    @pl.when(sid < 15)
    def _():
        pltpu.sync_copy(acc.at[pl.ds(sid * STR, STR)],
                        out_hbm.at[cid, pl.ds(sid * STR, STR)])
        pltpu.sync_copy(accden.at[pl.ds(sid * STR, STR)],
                        zbuf.at[pl.ds(0, STR)])
        pltpu.sync_copy(zbuf.at[pl.ds(0, STR)],
                        dens_hbm.at[pl.ds(cid * N + sid * STR, STR)])

    @pl.when(sid == 15)
    def _():
        pltpu.sync_copy(acc.at[pl.ds(15 * STR, STRL)],
                        out_hbm.at[cid, pl.ds(15 * STR, STRL)])
        pltpu.sync_copy(accden.at[pl.ds(15 * STR, STRL)], zbuf)
        pltpu.sync_copy(zbuf,
                        dens_hbm.at[pl.ds(cid * N + 15 * STR, STRL)])


def _scatter(xjs, ex, dst, zeros_hbm, zeros1_hbm):
    mesh = plsc.VectorSubcoreMesh(core_axis_name="c", subcore_axis_name="s",
                                  num_cores=NC, num_subcores=NS)
    f = pl.kernel(
        _scatter_body,
        out_type=[
            jax.ShapeDtypeStruct((NC, N, ROWW), jnp.float32),
            jax.ShapeDtypeStruct((NC * N,), jnp.float32),
        ],
        mesh=mesh,
        scratch_types=[
            pltpu.VMEM_SHARED((N, ROWW), jnp.float32),
            pltpu.VMEM_SHARED((N,), jnp.float32),
            pltpu.VMEM((1, CH), jnp.int32),
            pltpu.VMEM((CH, ROWW), jnp.float32),
            pltpu.VMEM((CH,), jnp.float32),
            pltpu.VMEM((STRL,), jnp.float32),
        ],
    )
    return f(xjs, ex, dst, zeros_hbm, zeros1_hbm)


def _gat_layer(x_pre, src, dst, ea, wet, att1, b, zeros_hbm, zeros1_hbm,
               Wl2, bl2, Wr2, br2, efm2, att2, relu_flag):
    xl, xr, a_self, selfmax = x_pre
    xj, xi = _gather(xl, xr, src, dst)
    alpha, amax = _alpha(xj, xi, ea, wet, att1)
    xjs, ex = _exscale(xj, alpha, amax, selfmax)
    nums, dens = _scatter(xjs, ex.reshape(E), dst, zeros_hbm, zeros1_hbm)
    return _combine(nums, dens.reshape(NC, N).T, xl, xr, a_self, amax,
                    selfmax, b, Wl2, bl2, Wr2, br2, efm2, att2, relu_flag)


def _mha(x, Win, bin_, Wout, bout, nh):
    L, Emb = x.shape
    hd = Emb // nh
    qkv = x @ Win.T + bin_
    q, k, v = jnp.split(qkv, 3, axis=-1)
    q = q.reshape(L, nh, hd).transpose(1, 0, 2)
    k = k.reshape(L, nh, hd).transpose(1, 0, 2)
    v = v.reshape(L, nh, hd).transpose(1, 0, 2)
    s = jnp.matmul(q, k.transpose(0, 2, 1)) * (1.0 / float(np.sqrt(hd)))
    a = jax.nn.softmax(s, axis=-1)
    o = jnp.matmul(a, v).transpose(1, 0, 2).reshape(L, Emb)
    return o @ Wout.T + bout


def _gru(hist, Wih, Whh, bih, bhh, hid):
    h = jnp.zeros((hid,), dtype=hist.dtype)
    outs = []
    for t in range(hist.shape[0]):
        gx = hist[t] @ Wih.T + bih
        gh = h @ Whh.T + bhh
        xr, xz, xn = jnp.split(gx, 3)
        hr, hz, hn = jnp.split(gh, 3)
        r = jax.nn.sigmoid(xr + hr)
        z = jax.nn.sigmoid(xz + hz)
        n = jnp.tanh(xn + r * hn)
        h = (1.0 - z) * n + z * h
        outs.append(h)
    return jnp.stack(outs, axis=0)


def kernel(x, edge_index, edge_attr, history, c1_Wl, c1_bl, c1_Wr, c1_br, c1_We, c1_att, c1_b, c2_Wl, c2_bl, c2_Wr, c2_br, c2_We, c2_att, c2_b, p_Win, p_bin, p_Wout, p_bout, m_W1, m_b1, m_W2, m_b2, g_Wih, g_Whh, g_bih, g_bhh, h_W1, h_b1, h_W2, h_b2):
    f32 = jnp.float32
    src = edge_index[0]
    dst = edge_index[1]

    colsum = _ea_colsum(edge_attr.reshape(E * EDIM // 128, 128))
    mean_ea = colsum[0].reshape(32, EDIM).sum(axis=0) * (1.0 / E)

    we1t = c1_We.T
    we2t = c2_We.T
    efm1 = (mean_ea @ we1t).reshape(1, HID)
    efm2 = (mean_ea @ we2t).reshape(1, HID)
    att1 = c1_att.reshape(1, HID)
    att2 = c2_att.reshape(1, HID)
    zeros_hbm = jnp.zeros((N, ROWW), f32)
    zeros1_hbm = jnp.zeros((N,), f32)
    one = jnp.ones((1, 1), f32)
    zero = jnp.zeros((1, 1), f32)

    pre1 = _pre(x, c1_Wl, c1_bl.reshape(1, HID), c1_Wr, c1_br.reshape(1, HID),
                efm1, att1)
    pre2 = _gat_layer(pre1, src, dst, edge_attr, we1t, att1,
                      c1_b.reshape(1, HID), zeros_hbm, zeros1_hbm,
                      c2_Wl, c2_bl.reshape(1, HID), c2_Wr,
                      c2_br.reshape(1, HID), efm2, att2, one)
    eye = jnp.eye(HID, dtype=f32)
    zb = jnp.zeros((1, HID), f32)
    h2 = _gat_layer(pre2, src, dst, edge_attr, we2t, att2,
                    c2_b.reshape(1, HID), zeros_hbm, zeros1_hbm,
                    eye, zb, eye, zb, zb, zb, zero)[0]

    a = _mha(h2, p_Win, p_bin, p_Wout, p_bout, 2)
    a = jnp.max(a, axis=0)
    m = jax.nn.leaky_relu(a.reshape(1, -1) @ m_W1.T + m_b1, 0.01)
    m = jax.nn.leaky_relu(m @ m_W2.T + m_b2, 0.01)
    g = _gru(history, g_Wih, g_Whh, g_bih, g_bhh, HID).reshape(1, -1)
    cat = jnp.concatenate([m, g], axis=1)
    o = jax.nn.leaky_relu(cat @ h_W1.T + h_b1, 0.01)
    return o @ h_W2.T + h_b2


# speedup vs baseline: 4.9003x; 1.1128x over previous
import functools

import jax
import jax.numpy as jnp
import numpy as np
from jax import lax
from jax.experimental import pallas as pl
from jax.experimental.pallas import tpu as pltpu
from jax.experimental.pallas import tpu_sc as plsc

N = 10000
E = 320000
HID = 128
EDIM = 4
ROWW = 128

NC, NS = 2, 16
NW = NC * NS
EPT = E // NW
CH = 80
NCHUNK = EPT // CH
NBLK = 10
BN = N // NBLK
EBLK = 125
BE = E // EBLK
STR = 624
STRL = N - 15 * STR

NEG = -1e30


def _eamean_body(ea_ref, o_ref):
    s = jnp.sum(ea_ref[...], axis=0)
    o_ref[...] = jnp.broadcast_to(s[None, :], (8, 128))


def _ea_colsum(ea_flat2d):
    return pl.pallas_call(
        _eamean_body,
        out_shape=jax.ShapeDtypeStruct((8, 128), jnp.float32),
    )(ea_flat2d)


def _pre_body(x_ref, wl_ref, bl_ref, wr_ref, br_ref, efm_ref, att_ref,
              xl_ref, xr_ref, as_ref, mx_ref):
    i = pl.program_id(0)
    x = x_ref[...]
    xl = jax.lax.dot_general(x, wl_ref[...], (((1,), (1,)), ((), ())),
                             preferred_element_type=jnp.float32) + bl_ref[...]
    xr = jax.lax.dot_general(x, wr_ref[...], (((1,), (1,)), ((), ())),
                             preferred_element_type=jnp.float32) + br_ref[...]
    xl_ref[...] = xl
    xr_ref[...] = xr
    u = xl + xr + efm_ref[...]
    u = jnp.where(u > 0, u, 0.2 * u)
    a_self = jnp.sum(u * att_ref[...], axis=1, keepdims=True)
    as_ref[...] = a_self

    @pl.when(i == 0)
    def _():
        mx_ref[...] = jnp.full((8, 128), NEG, jnp.float32)

    mx_ref[...] = jnp.maximum(mx_ref[...], jnp.max(a_self))


def _pre(x, Wl, bl, Wr, br, efm, att):
    return pl.pallas_call(
        _pre_body,
        grid=(NBLK,),
        in_specs=[
            pl.BlockSpec((BN, HID), lambda i: (i, 0)),
            pl.BlockSpec((HID, HID), lambda i: (0, 0)),
            pl.BlockSpec((1, HID), lambda i: (0, 0)),
            pl.BlockSpec((HID, HID), lambda i: (0, 0)),
            pl.BlockSpec((1, HID), lambda i: (0, 0)),
            pl.BlockSpec((1, HID), lambda i: (0, 0)),
            pl.BlockSpec((1, HID), lambda i: (0, 0)),
        ],
        out_specs=[
            pl.BlockSpec((BN, HID), lambda i: (i, 0)),
            pl.BlockSpec((BN, HID), lambda i: (i, 0)),
            pl.BlockSpec((BN, 1), lambda i: (i, 0)),
            pl.BlockSpec((8, 128), lambda i: (0, 0)),
        ],
        out_shape=[
            jax.ShapeDtypeStruct((N, HID), jnp.float32),
            jax.ShapeDtypeStruct((N, HID), jnp.float32),
            jax.ShapeDtypeStruct((N, 1), jnp.float32),
            jax.ShapeDtypeStruct((8, 128), jnp.float32),
        ],
    )(x, Wl, bl, Wr, br, efm, att)


def _alpha_body(xj_ref, xi_ref, ea_ref, wet_ref, att_ref, al_ref, mx_ref):
    i = pl.program_id(0)
    ef = jax.lax.dot_general(ea_ref[...], wet_ref[...], (((1,), (0,)), ((), ())),
                             preferred_element_type=jnp.float32)
    u = xj_ref[...] + xi_ref[...] + ef
    u = jnp.where(u > 0, u, 0.2 * u)
    a = jnp.sum(u * att_ref[...], axis=1, keepdims=True)
    al_ref[...] = a

    @pl.when(i == 0)
    def _():
        mx_ref[...] = jnp.full((8, 128), NEG, jnp.float32)

    mx_ref[...] = jnp.maximum(mx_ref[...], jnp.max(a))


def _alpha(xj, xi, ea, wet, att):
    return pl.pallas_call(
        _alpha_body,
        grid=(EBLK,),
        in_specs=[
            pl.BlockSpec((BE, HID), lambda i: (i, 0)),
            pl.BlockSpec((BE, HID), lambda i: (i, 0)),
            pl.BlockSpec((BE, EDIM), lambda i: (i, 0)),
            pl.BlockSpec((EDIM, HID), lambda i: (0, 0)),
            pl.BlockSpec((1, HID), lambda i: (0, 0)),
        ],
        out_specs=[
            pl.BlockSpec((BE, 1), lambda i: (i, 0)),
            pl.BlockSpec((8, 128), lambda i: (0, 0)),
        ],
        out_shape=[
            jax.ShapeDtypeStruct((E, 1), jnp.float32),
            jax.ShapeDtypeStruct((8, 128), jnp.float32),
        ],
    )(xj, xi, ea, wet, att)


def _exscale_body(xj_ref, al_ref, m1_ref, m2_ref, o_ref, ex_ref):
    m = jnp.maximum(jnp.max(m1_ref[...]), jnp.max(m2_ref[...]))
    ex = jnp.exp(al_ref[...] - m)
    o_ref[...] = xj_ref[...] * ex
    ex_ref[...] = ex


def _exscale(xj, alpha, m1, m2):
    return pl.pallas_call(
        _exscale_body,
        grid=(EBLK,),
        in_specs=[
            pl.BlockSpec((BE, HID), lambda i: (i, 0)),
            pl.BlockSpec((BE, 1), lambda i: (i, 0)),
            pl.BlockSpec((8, 128), lambda i: (0, 0)),
            pl.BlockSpec((8, 128), lambda i: (0, 0)),
        ],
        out_specs=[
            pl.BlockSpec((BE, ROWW), lambda i: (i, 0)),
            pl.BlockSpec((BE, 1), lambda i: (i, 0)),
        ],
        out_shape=[
            jax.ShapeDtypeStruct((E, ROWW), jnp.float32),
            jax.ShapeDtypeStruct((E, 1), jnp.float32),
        ],
    )(xj, alpha, m1, m2)


def _comb_body(nums_ref, dens_ref, xl_ref, xr_ref, as_ref, m1_ref, m2_ref,
               b_ref, wl2_ref, bl2_ref, wr2_ref, br2_ref, efm2_ref, att2_ref,
               relu_ref,
               xl2_ref, xr2_ref, as2_ref, mx2_ref):
    i = pl.program_id(0)
    m = jnp.maximum(jnp.max(m1_ref[...]), jnp.max(m2_ref[...]))
    exs = jnp.exp(as_ref[...] - m)
    xl = xl_ref[...]
    den = jnp.sum(dens_ref[...], axis=1, keepdims=True) + exs + 1e-16
    num = nums_ref[0] + nums_ref[1] + exs * xl
    h = num / den + b_ref[...]
    h = jnp.where(relu_ref[0, 0] > 0, jnp.maximum(h, 0.0), h)
    xl2 = jax.lax.dot_general(h, wl2_ref[...], (((1,), (1,)), ((), ())),
                              preferred_element_type=jnp.float32) + bl2_ref[...]
    xr2 = jax.lax.dot_general(h, wr2_ref[...], (((1,), (1,)), ((), ())),
                              preferred_element_type=jnp.float32) + br2_ref[...]
    xl2_ref[...] = xl2
    xr2_ref[...] = xr2
    u = xl2 + xr2 + efm2_ref[...]
    u = jnp.where(u > 0, u, 0.2 * u)
    a2 = jnp.sum(u * att2_ref[...], axis=1, keepdims=True)
    as2_ref[...] = a2

    @pl.when(i == 0)
    def _():
        mx2_ref[...] = jnp.full((8, 128), NEG, jnp.float32)

    mx2_ref[...] = jnp.maximum(mx2_ref[...], jnp.max(a2))


def _combine(nums, dens, xl, xr, a_self, m1, m2, b, Wl2, bl2, Wr2, br2, efm2,
             att2, relu_flag):
    return pl.pallas_call(
        _comb_body,
        grid=(NBLK,),
        in_specs=[
            pl.BlockSpec((2, BN, ROWW), lambda i: (0, i, 0)),
            pl.BlockSpec((BN, NC), lambda i: (i, 0)),
            pl.BlockSpec((BN, HID), lambda i: (i, 0)),
            pl.BlockSpec((BN, HID), lambda i: (i, 0)),
            pl.BlockSpec((BN, 1), lambda i: (i, 0)),
            pl.BlockSpec((8, 128), lambda i: (0, 0)),
            pl.BlockSpec((8, 128), lambda i: (0, 0)),
            pl.BlockSpec((1, HID), lambda i: (0, 0)),
            pl.BlockSpec((HID, HID), lambda i: (0, 0)),
            pl.BlockSpec((1, HID), lambda i: (0, 0)),
            pl.BlockSpec((HID, HID), lambda i: (0, 0)),
            pl.BlockSpec((1, HID), lambda i: (0, 0)),
            pl.BlockSpec((1, HID), lambda i: (0, 0)),
            pl.BlockSpec((1, HID), lambda i: (0, 0)),
            pl.BlockSpec((1, 1), lambda i: (0, 0), memory_space=pltpu.SMEM),
        ],
        out_specs=[
            pl.BlockSpec((BN, HID), lambda i: (i, 0)),
            pl.BlockSpec((BN, HID), lambda i: (i, 0)),
            pl.BlockSpec((BN, 1), lambda i: (i, 0)),
            pl.BlockSpec((8, 128), lambda i: (0, 0)),
        ],
        out_shape=[
            jax.ShapeDtypeStruct((N, HID), jnp.float32),
            jax.ShapeDtypeStruct((N, HID), jnp.float32),
            jax.ShapeDtypeStruct((N, 1), jnp.float32),
            jax.ShapeDtypeStruct((8, 128), jnp.float32),
        ],
    )(nums, dens, xl, xr, a_self, m1, m2, b, Wl2, bl2, Wr2, br2, efm2, att2,
      relu_flag)


def _gather_body(xl_hbm, xr_hbm, src_hbm, dst_hbm, xj_hbm, xi_hbm,
                 sidx, didx, bj, bi, sem):
    cid = lax.axis_index("c")
    sid = lax.axis_index("s")
    wid = sid * NC + cid
    base0 = wid * EPT

    def step(c, _):
        base = base0 + c * CH
        pltpu.sync_copy(src_hbm.at[pl.ds(base, CH)], sidx.at[0])
        pltpu.sync_copy(dst_hbm.at[pl.ds(base, CH)], didx.at[0])
        cj = pltpu.async_copy(xl_hbm.at[sidx.at[0]], bj, sem)
        ci = pltpu.async_copy(xr_hbm.at[didx.at[0]], bi, sem)
        cj.wait()
        ci.wait()
        pltpu.sync_copy(bj, xj_hbm.at[pl.ds(base, CH)])
        pltpu.sync_copy(bi, xi_hbm.at[pl.ds(base, CH)])
        return _

    lax.fori_loop(0, NCHUNK, step, 0)


def _gather(xl, xr, src, dst):
    mesh = plsc.VectorSubcoreMesh(core_axis_name="c", subcore_axis_name="s",
                                  num_cores=NC, num_subcores=NS)
    f = pl.kernel(
        _gather_body,
        out_type=[
            jax.ShapeDtypeStruct((E, HID), jnp.float32),
            jax.ShapeDtypeStruct((E, HID), jnp.float32),
        ],
        mesh=mesh,
        scratch_types=[
            pltpu.VMEM((1, CH), jnp.int32),
            pltpu.VMEM((1, CH), jnp.int32),
            pltpu.VMEM((CH, HID), jnp.float32),
            pltpu.VMEM((CH, HID), jnp.float32),
            pltpu.SemaphoreType.DMA,
        ],
    )
    return f(xl, xr, src, dst)


def _scatter_body(xjs_hbm, ex_hbm, dst_hbm, zeros_hbm, zeros1_hbm,
                  out_hbm, dens_hbm, acc, accden, didx, rows, exbuf, zbuf):
    cid = lax.axis_index("c")
    sid = lax.axis_index("s")
    base0 = (sid * NC + cid) * EPT

    pltpu.sync_copy(zeros1_hbm.at[pl.ds(0, STRL)], zbuf)

    @pl.when(sid < 15)
    def _():
        pltpu.sync_copy(zeros_hbm.at[pl.ds(sid * STR, STR)],
                        acc.at[pl.ds(sid * STR, STR)])
        pltpu.sync_copy(zbuf.at[pl.ds(0, STR)],
                        accden.at[pl.ds(sid * STR, STR)])

    @pl.when(sid == 15)
    def _():
        pltpu.sync_copy(zeros_hbm.at[pl.ds(15 * STR, STRL)],
                        acc.at[pl.ds(15 * STR, STRL)])
        pltpu.sync_copy(zbuf,
                        accden.at[pl.ds(15 * STR, STRL)])

    plsc.subcore_barrier()

    def step(c, _):
        base = base0 + c * CH
        pltpu.sync_copy(dst_hbm.at[pl.ds(base, CH)], didx.at[0])
        pltpu.sync_copy(xjs_hbm.at[pl.ds(base, CH)], rows)
        pltpu.sync_copy(ex_hbm.at[pl.ds(base, CH)], exbuf)
        pltpu.sync_copy(rows, acc.at[didx.at[0]], add=True)
        pltpu.sync_copy(exbuf, accden.at[didx.at[0]], add=True)
        return _

    lax.fori_loop(0, NCHUNK, step, 0)
    plsc.subcore_barrier()

    @pl.when(sid < 15)
    def _():
        pltpu.sync_copy(acc.at[pl.ds(sid * STR, STR)],
                        out_hbm.at[cid, pl.ds(sid * STR, STR)])
        pltpu.sync_copy(accden.at[pl.ds(sid * STR, STR)],
                        zbuf.at[pl.ds(0, STR)])
        pltpu.sync_copy(zbuf.at[pl.ds(0, STR)],
                        dens_hbm.at[pl.ds(cid * N + sid * STR, STR)])

    @pl.when(sid == 15)
    def _():
        pltpu.sync_copy(acc.at[pl.ds(15 * STR, STRL)],
                        out_hbm.at[cid, pl.ds(15 * STR, STRL)])
        pltpu.sync_copy(accden.at[pl.ds(15 * STR, STRL)], zbuf)
        pltpu.sync_copy(zbuf,
                        dens_hbm.at[pl.ds(cid * N + 15 * STR, STRL)])


def _scatter(xjs, ex, dst, zeros_hbm, zeros1_hbm):
    mesh = plsc.VectorSubcoreMesh(core_axis_name="c", subcore_axis_name="s",
                                  num_cores=NC, num_subcores=NS)
    f = pl.kernel(
        _scatter_body,
        out_type=[
            jax.ShapeDtypeStruct((NC, N, ROWW), jnp.float32),
            jax.ShapeDtypeStruct((NC * N,), jnp.float32),
        ],
        mesh=mesh,
        scratch_types=[
            pltpu.VMEM_SHARED((N, ROWW), jnp.float32),
            pltpu.VMEM_SHARED((N,), jnp.float32),
            pltpu.VMEM((1, CH), jnp.int32),
            pltpu.VMEM((CH, ROWW), jnp.float32),
            pltpu.VMEM((CH,), jnp.float32),
            pltpu.VMEM((STRL,), jnp.float32),
        ],
    )
    return f(xjs, ex, dst, zeros_hbm, zeros1_hbm)


def _gat_layer(x_pre, src, dst, ea, wet, att1, b, zeros_hbm, zeros1_hbm,
               Wl2, bl2, Wr2, br2, efm2, att2, relu_flag):
    xl, xr, a_self, selfmax = x_pre
    xj, xi = _gather(xl, xr, src, dst)
    alpha, amax = _alpha(xj, xi, ea, wet, att1)
    xjs, ex = _exscale(xj, alpha, amax, selfmax)
    nums, dens = _scatter(xjs, ex.reshape(E), dst, zeros_hbm, zeros1_hbm)
    return _combine(nums, dens.reshape(NC, N).T, xl, xr, a_self, amax,
                    selfmax, b, Wl2, bl2, Wr2, br2, efm2, att2, relu_flag)


BQ = 200
NQB = N // BQ


def _qkv_body(x_ref, w_ref, b_ref, o_ref):
    o_ref[...] = jax.lax.dot_general(
        x_ref[...], w_ref[...], (((1,), (0,)), ((), ())),
        preferred_element_type=jnp.float32) + b_ref[...]


def _qkv(x, winT, bin_):
    return pl.pallas_call(
        _qkv_body,
        grid=(NBLK,),
        in_specs=[
            pl.BlockSpec((BN, HID), lambda i: (i, 0)),
            pl.BlockSpec((HID, 3 * HID), lambda i: (0, 0)),
            pl.BlockSpec((1, 3 * HID), lambda i: (0, 0)),
        ],
        out_specs=pl.BlockSpec((BN, 3 * HID), lambda i: (i, 0)),
        out_shape=jax.ShapeDtypeStruct((N, 3 * HID), jnp.float32),
    )(x, winT, bin_)


def _mhamax_body(q_ref, kv_ref, woutT_ref, bout_ref, mx_ref):
    i = pl.program_id(0)
    qq = q_ref[...]
    outs = []
    for h in range(2):
        q = qq[:, h * 64:(h + 1) * 64] * 0.125
        k = kv_ref[:, HID + h * 64:HID + (h + 1) * 64]
        v = kv_ref[:, 2 * HID + h * 64:2 * HID + (h + 1) * 64]
        s = jax.lax.dot_general(q, k, (((1,), (1,)), ((), ())),
                                preferred_element_type=jnp.float32)
        m = jnp.max(s, axis=1, keepdims=True)
        p = jnp.exp(s - m)
        l = jnp.sum(p, axis=1, keepdims=True)
        o = jax.lax.dot_general(p, v, (((1,), (0,)), ((), ())),
                                preferred_element_type=jnp.float32) / l
        outs.append(o)
    o = jnp.concatenate(outs, axis=1)
    oo = jax.lax.dot_general(o, woutT_ref[...], (((1,), (0,)), ((), ())),
                             preferred_element_type=jnp.float32) + bout_ref[...]

    @pl.when(i == 0)
    def _():
        mx_ref[...] = jnp.full((8, 128), NEG, jnp.float32)

    mx_ref[...] = jnp.maximum(mx_ref[...], jnp.max(oo, axis=0, keepdims=True))


def _mha_max(h2, Win, bin_, Wout, bout):
    qkv = _qkv(h2, Win.T, bin_.reshape(1, 3 * HID))
    return pl.pallas_call(
        _mhamax_body,
        grid=(NQB,),
        in_specs=[
            pl.BlockSpec((BQ, 3 * HID), lambda i: (i, 0)),
            pl.BlockSpec((N, 3 * HID), lambda i: (0, 0)),
            pl.BlockSpec((HID, HID), lambda i: (0, 0)),
            pl.BlockSpec((1, HID), lambda i: (0, 0)),
        ],
        out_specs=pl.BlockSpec((8, 128), lambda i: (0, 0)),
        out_shape=jax.ShapeDtypeStruct((8, 128), jnp.float32),
    )(qkv, qkv, Wout.T, bout.reshape(1, HID))


def _gru(hist, Wih, Whh, bih, bhh, hid):
    h = jnp.zeros((hid,), dtype=hist.dtype)
    outs = []
    for t in range(hist.shape[0]):
        gx = hist[t] @ Wih.T + bih
        gh = h @ Whh.T + bhh
        xr, xz, xn = jnp.split(gx, 3)
        hr, hz, hn = jnp.split(gh, 3)
        r = jax.nn.sigmoid(xr + hr)
        z = jax.nn.sigmoid(xz + hz)
        n = jnp.tanh(xn + r * hn)
        h = (1.0 - z) * n + z * h
        outs.append(h)
    return jnp.stack(outs, axis=0)


def kernel(x, edge_index, edge_attr, history, c1_Wl, c1_bl, c1_Wr, c1_br, c1_We, c1_att, c1_b, c2_Wl, c2_bl, c2_Wr, c2_br, c2_We, c2_att, c2_b, p_Win, p_bin, p_Wout, p_bout, m_W1, m_b1, m_W2, m_b2, g_Wih, g_Whh, g_bih, g_bhh, h_W1, h_b1, h_W2, h_b2):
    f32 = jnp.float32
    src = edge_index[0]
    dst = edge_index[1]

    colsum = _ea_colsum(edge_attr.reshape(E * EDIM // 128, 128))
    mean_ea = colsum[0].reshape(32, EDIM).sum(axis=0) * (1.0 / E)

    we1t = c1_We.T
    we2t = c2_We.T
    efm1 = (mean_ea @ we1t).reshape(1, HID)
    efm2 = (mean_ea @ we2t).reshape(1, HID)
    att1 = c1_att.reshape(1, HID)
    att2 = c2_att.reshape(1, HID)
    zeros_hbm = jnp.zeros((N, ROWW), f32)
    zeros1_hbm = jnp.zeros((N,), f32)
    one = jnp.ones((1, 1), f32)
    zero = jnp.zeros((1, 1), f32)

    pre1 = _pre(x, c1_Wl, c1_bl.reshape(1, HID), c1_Wr, c1_br.reshape(1, HID),
                efm1, att1)
    pre2 = _gat_layer(pre1, src, dst, edge_attr, we1t, att1,
                      c1_b.reshape(1, HID), zeros_hbm, zeros1_hbm,
                      c2_Wl, c2_bl.reshape(1, HID), c2_Wr,
                      c2_br.reshape(1, HID), efm2, att2, one)
    eye = jnp.eye(HID, dtype=f32)
    zb = jnp.zeros((1, HID), f32)
    h2 = _gat_layer(pre2, src, dst, edge_attr, we2t, att2,
                    c2_b.reshape(1, HID), zeros_hbm, zeros1_hbm,
                    eye, zb, eye, zb, zb, zb, zero)[0]

    a = jnp.max(_mha_max(h2, p_Win, p_bin, p_Wout, p_bout), axis=0)
    m = jax.nn.leaky_relu(a.reshape(1, -1) @ m_W1.T + m_b1, 0.01)
    m = jax.nn.leaky_relu(m @ m_W2.T + m_b2, 0.01)
    g = _gru(history, g_Wih, g_Whh, g_bih, g_bhh, HID).reshape(1, -1)
    cat = jnp.concatenate([m, g], axis=1)
    o = jax.nn.leaky_relu(cat @ h_W1.T + h_b1, 0.01)
    return o @ h_W2.T + h_b2


# speedup vs baseline: 5.6899x; 1.1612x over previous
import functools

import jax
import jax.numpy as jnp
import numpy as np
from jax import lax
from jax.experimental import pallas as pl
from jax.experimental.pallas import tpu as pltpu
from jax.experimental.pallas import tpu_sc as plsc

N = 10000
E = 320000
HID = 128
EDIM = 4
ROWW = 128

NC, NS = 2, 16
NW = NC * NS
EPT = E // NW
CH = 80
NCHUNK = EPT // CH
NBLK = 10
BN = N // NBLK
EBLK = 125
BE = E // EBLK
STR = 624
STRL = N - 15 * STR

NEG = -1e30


def _eamean_body(ea_ref, o_ref):
    s = jnp.sum(ea_ref[...], axis=0)
    o_ref[...] = jnp.broadcast_to(s[None, :], (8, 128))


def _ea_colsum(ea_flat2d):
    return pl.pallas_call(
        _eamean_body,
        out_shape=jax.ShapeDtypeStruct((8, 128), jnp.float32),
    )(ea_flat2d)


def _pre_body(x_ref, wl_ref, bl_ref, wr_ref, br_ref, efm_ref, att_ref,
              xl_ref, xr_ref, as_ref, mx_ref):
    i = pl.program_id(0)
    x = x_ref[...]
    xl = jax.lax.dot_general(x, wl_ref[...], (((1,), (1,)), ((), ())),
                             preferred_element_type=jnp.float32) + bl_ref[...]
    xr = jax.lax.dot_general(x, wr_ref[...], (((1,), (1,)), ((), ())),
                             preferred_element_type=jnp.float32) + br_ref[...]
    xl_ref[...] = xl
    xr_ref[...] = xr
    u = xl + xr + efm_ref[...]
    u = jnp.where(u > 0, u, 0.2 * u)
    a_self = jnp.sum(u * att_ref[...], axis=1, keepdims=True)
    as_ref[...] = a_self

    @pl.when(i == 0)
    def _():
        mx_ref[...] = jnp.full((8, 128), NEG, jnp.float32)

    mx_ref[...] = jnp.maximum(mx_ref[...], jnp.max(a_self))


def _pre(x, Wl, bl, Wr, br, efm, att):
    return pl.pallas_call(
        _pre_body,
        grid=(NBLK,),
        in_specs=[
            pl.BlockSpec((BN, HID), lambda i: (i, 0)),
            pl.BlockSpec((HID, HID), lambda i: (0, 0)),
            pl.BlockSpec((1, HID), lambda i: (0, 0)),
            pl.BlockSpec((HID, HID), lambda i: (0, 0)),
            pl.BlockSpec((1, HID), lambda i: (0, 0)),
            pl.BlockSpec((1, HID), lambda i: (0, 0)),
            pl.BlockSpec((1, HID), lambda i: (0, 0)),
        ],
        out_specs=[
            pl.BlockSpec((BN, HID), lambda i: (i, 0)),
            pl.BlockSpec((BN, HID), lambda i: (i, 0)),
            pl.BlockSpec((BN, 1), lambda i: (i, 0)),
            pl.BlockSpec((8, 128), lambda i: (0, 0)),
        ],
        out_shape=[
            jax.ShapeDtypeStruct((N, HID), jnp.float32),
            jax.ShapeDtypeStruct((N, HID), jnp.float32),
            jax.ShapeDtypeStruct((N, 1), jnp.float32),
            jax.ShapeDtypeStruct((8, 128), jnp.float32),
        ],
    )(x, Wl, bl, Wr, br, efm, att)


def _alpha_body(xj_ref, xi_ref, ea_ref, wet_ref, att_ref, al_ref, mx_ref):
    i = pl.program_id(0)
    ef = jax.lax.dot_general(ea_ref[...], wet_ref[...], (((1,), (0,)), ((), ())),
                             preferred_element_type=jnp.float32)
    u = xj_ref[...] + xi_ref[...] + ef
    u = jnp.where(u > 0, u, 0.2 * u)
    a = jnp.sum(u * att_ref[...], axis=1, keepdims=True)
    al_ref[...] = a

    @pl.when(i == 0)
    def _():
        mx_ref[...] = jnp.full((8, 128), NEG, jnp.float32)

    mx_ref[...] = jnp.maximum(mx_ref[...], jnp.max(a))


def _alpha(xj, xi, ea, wet, att):
    return pl.pallas_call(
        _alpha_body,
        grid=(EBLK,),
        in_specs=[
            pl.BlockSpec((BE, HID), lambda i: (i, 0)),
            pl.BlockSpec((BE, HID), lambda i: (i, 0)),
            pl.BlockSpec((BE, EDIM), lambda i: (i, 0)),
            pl.BlockSpec((EDIM, HID), lambda i: (0, 0)),
            pl.BlockSpec((1, HID), lambda i: (0, 0)),
        ],
        out_specs=[
            pl.BlockSpec((BE, 1), lambda i: (i, 0)),
            pl.BlockSpec((8, 128), lambda i: (0, 0)),
        ],
        out_shape=[
            jax.ShapeDtypeStruct((E, 1), jnp.float32),
            jax.ShapeDtypeStruct((8, 128), jnp.float32),
        ],
    )(xj, xi, ea, wet, att)


def _exscale_body(xj_ref, al_ref, m1_ref, m2_ref, o_ref, ex_ref):
    m = jnp.maximum(jnp.max(m1_ref[...]), jnp.max(m2_ref[...]))
    ex = jnp.exp(al_ref[...] - m)
    o_ref[...] = xj_ref[...] * ex
    ex_ref[...] = ex


def _exscale(xj, alpha, m1, m2):
    return pl.pallas_call(
        _exscale_body,
        grid=(EBLK,),
        in_specs=[
            pl.BlockSpec((BE, HID), lambda i: (i, 0)),
            pl.BlockSpec((BE, 1), lambda i: (i, 0)),
            pl.BlockSpec((8, 128), lambda i: (0, 0)),
            pl.BlockSpec((8, 128), lambda i: (0, 0)),
        ],
        out_specs=[
            pl.BlockSpec((BE, ROWW), lambda i: (i, 0)),
            pl.BlockSpec((BE, 1), lambda i: (i, 0)),
        ],
        out_shape=[
            jax.ShapeDtypeStruct((E, ROWW), jnp.float32),
            jax.ShapeDtypeStruct((E, 1), jnp.float32),
        ],
    )(xj, alpha, m1, m2)


def _comb_body(nums_ref, dens_ref, xl_ref, xr_ref, as_ref, m1_ref, m2_ref,
               b_ref, wl2_ref, bl2_ref, wr2_ref, br2_ref, efm2_ref, att2_ref,
               relu_ref,
               xl2_ref, xr2_ref, as2_ref, mx2_ref):
    i = pl.program_id(0)
    m = jnp.maximum(jnp.max(m1_ref[...]), jnp.max(m2_ref[...]))
    exs = jnp.exp(as_ref[...] - m)
    xl = xl_ref[...]
    den = jnp.sum(dens_ref[...], axis=1, keepdims=True) + exs + 1e-16
    num = nums_ref[0] + nums_ref[1] + exs * xl
    h = num / den + b_ref[...]
    h = jnp.where(relu_ref[0, 0] > 0, jnp.maximum(h, 0.0), h)
    xl2 = jax.lax.dot_general(h, wl2_ref[...], (((1,), (1,)), ((), ())),
                              preferred_element_type=jnp.float32) + bl2_ref[...]
    xr2 = jax.lax.dot_general(h, wr2_ref[...], (((1,), (1,)), ((), ())),
                              preferred_element_type=jnp.float32) + br2_ref[...]
    xl2_ref[...] = xl2
    xr2_ref[...] = xr2
    u = xl2 + xr2 + efm2_ref[...]
    u = jnp.where(u > 0, u, 0.2 * u)
    a2 = jnp.sum(u * att2_ref[...], axis=1, keepdims=True)
    as2_ref[...] = a2

    @pl.when(i == 0)
    def _():
        mx2_ref[...] = jnp.full((8, 128), NEG, jnp.float32)

    mx2_ref[...] = jnp.maximum(mx2_ref[...], jnp.max(a2))


def _combine(nums, dens, xl, xr, a_self, m1, m2, b, Wl2, bl2, Wr2, br2, efm2,
             att2, relu_flag):
    return pl.pallas_call(
        _comb_body,
        grid=(NBLK,),
        in_specs=[
            pl.BlockSpec((2, BN, ROWW), lambda i: (0, i, 0)),
            pl.BlockSpec((BN, NC), lambda i: (i, 0)),
            pl.BlockSpec((BN, HID), lambda i: (i, 0)),
            pl.BlockSpec((BN, HID), lambda i: (i, 0)),
            pl.BlockSpec((BN, 1), lambda i: (i, 0)),
            pl.BlockSpec((8, 128), lambda i: (0, 0)),
            pl.BlockSpec((8, 128), lambda i: (0, 0)),
            pl.BlockSpec((1, HID), lambda i: (0, 0)),
            pl.BlockSpec((HID, HID), lambda i: (0, 0)),
            pl.BlockSpec((1, HID), lambda i: (0, 0)),
            pl.BlockSpec((HID, HID), lambda i: (0, 0)),
            pl.BlockSpec((1, HID), lambda i: (0, 0)),
            pl.BlockSpec((1, HID), lambda i: (0, 0)),
            pl.BlockSpec((1, HID), lambda i: (0, 0)),
            pl.BlockSpec((1, 1), lambda i: (0, 0), memory_space=pltpu.SMEM),
        ],
        out_specs=[
            pl.BlockSpec((BN, HID), lambda i: (i, 0)),
            pl.BlockSpec((BN, HID), lambda i: (i, 0)),
            pl.BlockSpec((BN, 1), lambda i: (i, 0)),
            pl.BlockSpec((8, 128), lambda i: (0, 0)),
        ],
        out_shape=[
            jax.ShapeDtypeStruct((N, HID), jnp.float32),
            jax.ShapeDtypeStruct((N, HID), jnp.float32),
            jax.ShapeDtypeStruct((N, 1), jnp.float32),
            jax.ShapeDtypeStruct((8, 128), jnp.float32),
        ],
    )(nums, dens, xl, xr, a_self, m1, m2, b, Wl2, bl2, Wr2, br2, efm2, att2,
      relu_flag)


GB = 5


def _gather_body(xl_hbm, xr_hbm, src_hbm, dst_hbm, xj_hbm, xi_hbm, *scr):
    cid = lax.axis_index("c")
    sid = lax.axis_index("s")
    wid = sid * NC + cid
    base0 = wid * EPT
    ii = scr[0:GB]
    bjs = scr[GB:2 * GB]
    bis = scr[2 * GB:3 * GB]
    gsem = scr[3 * GB:4 * GB]
    ssem = scr[4 * GB:5 * GB]

    def loadidx(c, b):
        base = base0 + c * CH
        pltpu.sync_copy(src_hbm.at[pl.ds(base, CH)], ii[b].at[0])
        pltpu.sync_copy(dst_hbm.at[pl.ds(base, CH)], ii[b].at[1])

    def startg(b):
        pltpu.async_copy(xl_hbm.at[ii[b].at[0]], bjs[b], gsem[b])
        pltpu.async_copy(xr_hbm.at[ii[b].at[1]], bis[b], gsem[b])

    def waitg(b):
        pltpu.make_async_copy(xl_hbm.at[ii[b].at[0]], bjs[b], gsem[b]).wait()
        pltpu.make_async_copy(xr_hbm.at[ii[b].at[1]], bis[b], gsem[b]).wait()

    def starts(c, b):
        base = base0 + c * CH
        pltpu.async_copy(bjs[b], xj_hbm.at[pl.ds(base, CH)], ssem[b])
        pltpu.async_copy(bis[b], xi_hbm.at[pl.ds(base, CH)], ssem[b])

    def waits(c, b):
        base = base0 + c * CH
        pltpu.make_async_copy(bjs[b], xj_hbm.at[pl.ds(base, CH)],
                              ssem[b]).wait()
        pltpu.make_async_copy(bis[b], xi_hbm.at[pl.ds(base, CH)],
                              ssem[b]).wait()

    for b in range(GB - 1):
        loadidx(b, b)
        startg(b)

    def outer(g, carry):
        g4 = g * GB
        for b in range(GB):
            c = g4 + b
            waitg(b)
            starts(c, b)
            pb = (b - 1) % GB
            if b == 0:
                @pl.when(c >= 1)
                def _():
                    waits(c - 1, pb)
            else:
                waits(c - 1, pb)

            @pl.when(c + GB - 1 < NCHUNK)
            def _():
                loadidx(c + GB - 1, pb)
                startg(pb)
        return carry

    lax.fori_loop(0, NCHUNK // GB, outer, 0)
    waits(NCHUNK - 1, (NCHUNK - 1) % GB)


def _gather(xl, xr, src, dst):
    mesh = plsc.VectorSubcoreMesh(core_axis_name="c", subcore_axis_name="s",
                                  num_cores=NC, num_subcores=NS)
    f = pl.kernel(
        _gather_body,
        out_type=[
            jax.ShapeDtypeStruct((E, HID), jnp.float32),
            jax.ShapeDtypeStruct((E, HID), jnp.float32),
        ],
        mesh=mesh,
        scratch_types=(
            [pltpu.VMEM((2, CH), jnp.int32) for _ in range(GB)]
            + [pltpu.VMEM((CH, HID), jnp.float32) for _ in range(2 * GB)]
            + [pltpu.SemaphoreType.DMA for _ in range(2 * GB)]
        ),
    )
    return f(xl, xr, src, dst)


SB = 2
SCH = 40
SNCHUNK = EPT // SCH


def _scatter_body(xjs_hbm, ex_hbm, dst_hbm, zeros_hbm, zeros1_hbm,
                  out_hbm, dens_hbm, acc, accden, zbuf, *scr):
    cid = lax.axis_index("c")
    sid = lax.axis_index("s")
    base0 = (sid * NC + cid) * EPT
    dd = scr[0:SB]
    rr = scr[SB:2 * SB]
    ee = scr[2 * SB:3 * SB]
    ll = scr[3 * SB:4 * SB]

    def startl(c, b):
        base = base0 + c * SCH
        pltpu.async_copy(dst_hbm.at[pl.ds(base, SCH)], dd[b].at[0], ll[b])
        pltpu.async_copy(xjs_hbm.at[pl.ds(base, SCH)], rr[b], ll[b])
        pltpu.async_copy(ex_hbm.at[pl.ds(base, SCH)], ee[b], ll[b])

    def waitl(c, b):
        base = base0 + c * SCH
        pltpu.make_async_copy(dst_hbm.at[pl.ds(base, SCH)], dd[b].at[0],
                              ll[b]).wait()
        pltpu.make_async_copy(xjs_hbm.at[pl.ds(base, SCH)], rr[b],
                              ll[b]).wait()
        pltpu.make_async_copy(ex_hbm.at[pl.ds(base, SCH)], ee[b],
                              ll[b]).wait()

    pltpu.sync_copy(zeros1_hbm.at[pl.ds(0, STRL)], zbuf)

    @pl.when(sid < 15)
    def _():
        pltpu.sync_copy(zeros_hbm.at[pl.ds(sid * STR, STR)],
                        acc.at[pl.ds(sid * STR, STR)])
        pltpu.sync_copy(zbuf.at[pl.ds(0, STR)],
                        accden.at[pl.ds(sid * STR, STR)])

    @pl.when(sid == 15)
    def _():
        pltpu.sync_copy(zeros_hbm.at[pl.ds(15 * STR, STRL)],
                        acc.at[pl.ds(15 * STR, STRL)])
        pltpu.sync_copy(zbuf,
                        accden.at[pl.ds(15 * STR, STRL)])

    plsc.subcore_barrier()

    def outer(g, carry):
        gs = g * SB
        for b in range(SB):
            c = gs + b
            startl(c, b)
            waitl(c, b)
            pltpu.sync_copy(rr[b], acc.at[dd[b].at[0]], add=True)
            pltpu.sync_copy(ee[b], accden.at[dd[b].at[0]], add=True)
        return carry

    lax.fori_loop(0, SNCHUNK // SB, outer, 0)
    plsc.subcore_barrier()

    @pl.when(sid < 15)
    def _():
        pltpu.sync_copy(acc.at[pl.ds(sid * STR, STR)],
                        out_hbm.at[cid, pl.ds(sid * STR, STR)])
        pltpu.sync_copy(accden.at[pl.ds(sid * STR, STR)],
                        zbuf.at[pl.ds(0, STR)])
        pltpu.sync_copy(zbuf.at[pl.ds(0, STR)],
                        dens_hbm.at[pl.ds(cid * N + sid * STR, STR)])

    @pl.when(sid == 15)
    def _():
        pltpu.sync_copy(acc.at[pl.ds(15 * STR, STRL)],
                        out_hbm.at[cid, pl.ds(15 * STR, STRL)])
        pltpu.sync_copy(accden.at[pl.ds(15 * STR, STRL)], zbuf)
        pltpu.sync_copy(zbuf,
                        dens_hbm.at[pl.ds(cid * N + 15 * STR, STRL)])


def _scatter(xjs, ex, dst, zeros_hbm, zeros1_hbm):
    mesh = plsc.VectorSubcoreMesh(core_axis_name="c", subcore_axis_name="s",
                                  num_cores=NC, num_subcores=NS)
    f = pl.kernel(
        _scatter_body,
        out_type=[
            jax.ShapeDtypeStruct((NC, N, ROWW), jnp.float32),
            jax.ShapeDtypeStruct((NC * N,), jnp.float32),
        ],
        mesh=mesh,
        scratch_types=(
            [
                pltpu.VMEM_SHARED((N, ROWW), jnp.float32),
                pltpu.VMEM_SHARED((N,), jnp.float32),
                pltpu.VMEM((STRL,), jnp.float32),
            ]
            + [pltpu.VMEM((1, SCH), jnp.int32) for _ in range(SB)]
            + [pltpu.VMEM((SCH, ROWW), jnp.float32) for _ in range(SB)]
            + [pltpu.VMEM((SCH,), jnp.float32) for _ in range(SB)]
            + [pltpu.SemaphoreType.DMA for _ in range(SB)]
        ),
    )
    return f(xjs, ex, dst, zeros_hbm, zeros1_hbm)


def _gat_layer(x_pre, src, dst, ea, wet, att1, b, zeros_hbm, zeros1_hbm,
               Wl2, bl2, Wr2, br2, efm2, att2, relu_flag):
    xl, xr, a_self, selfmax = x_pre
    xj, xi = _gather(xl, xr, src, dst)
    alpha, amax = _alpha(xj, xi, ea, wet, att1)
    xjs, ex = _exscale(xj, alpha, amax, selfmax)
    nums, dens = _scatter(xjs, ex.reshape(E), dst, zeros_hbm, zeros1_hbm)
    return _combine(nums, dens.reshape(NC, N).T, xl, xr, a_self, amax,
                    selfmax, b, Wl2, bl2, Wr2, br2, efm2, att2, relu_flag)


BQ = 200
NQB = N // BQ


def _qkv_body(x_ref, w_ref, b_ref, o_ref):
    o_ref[...] = jax.lax.dot_general(
        x_ref[...], w_ref[...], (((1,), (0,)), ((), ())),
        preferred_element_type=jnp.float32) + b_ref[...]


def _qkv(x, winT, bin_):
    return pl.pallas_call(
        _qkv_body,
        grid=(NBLK,),
        in_specs=[
            pl.BlockSpec((BN, HID), lambda i: (i, 0)),
            pl.BlockSpec((HID, 3 * HID), lambda i: (0, 0)),
            pl.BlockSpec((1, 3 * HID), lambda i: (0, 0)),
        ],
        out_specs=pl.BlockSpec((BN, 3 * HID), lambda i: (i, 0)),
        out_shape=jax.ShapeDtypeStruct((N, 3 * HID), jnp.float32),
    )(x, winT, bin_)


def _mhamax_body(q_ref, kv_ref, woutT_ref, bout_ref, mx_ref):
    i = pl.program_id(0)
    qq = q_ref[...]
    outs = []
    for h in range(2):
        q = qq[:, h * 64:(h + 1) * 64] * 0.125
        k = kv_ref[:, HID + h * 64:HID + (h + 1) * 64]
        v = kv_ref[:, 2 * HID + h * 64:2 * HID + (h + 1) * 64]
        s = jax.lax.dot_general(q, k, (((1,), (1,)), ((), ())),
                                preferred_element_type=jnp.float32)
        m = jnp.max(s, axis=1, keepdims=True)
        p = jnp.exp(s - m)
        l = jnp.sum(p, axis=1, keepdims=True)
        o = jax.lax.dot_general(p, v, (((1,), (0,)), ((), ())),
                                preferred_element_type=jnp.float32) / l
        outs.append(o)
    o = jnp.concatenate(outs, axis=1)
    oo = jax.lax.dot_general(o, woutT_ref[...], (((1,), (0,)), ((), ())),
                             preferred_element_type=jnp.float32) + bout_ref[...]

    @pl.when(i == 0)
    def _():
        mx_ref[...] = jnp.full((8, 128), NEG, jnp.float32)

    mx_ref[...] = jnp.maximum(mx_ref[...], jnp.max(oo, axis=0, keepdims=True))


def _mha_max(h2, Win, bin_, Wout, bout):
    qkv = _qkv(h2, Win.T, bin_.reshape(1, 3 * HID))
    return pl.pallas_call(
        _mhamax_body,
        grid=(NQB,),
        in_specs=[
            pl.BlockSpec((BQ, 3 * HID), lambda i: (i, 0)),
            pl.BlockSpec((N, 3 * HID), lambda i: (0, 0)),
            pl.BlockSpec((HID, HID), lambda i: (0, 0)),
            pl.BlockSpec((1, HID), lambda i: (0, 0)),
        ],
        out_specs=pl.BlockSpec((8, 128), lambda i: (0, 0)),
        out_shape=jax.ShapeDtypeStruct((8, 128), jnp.float32),
    )(qkv, qkv, Wout.T, bout.reshape(1, HID))


def _gru(hist, Wih, Whh, bih, bhh, hid):
    h = jnp.zeros((hid,), dtype=hist.dtype)
    outs = []
    for t in range(hist.shape[0]):
        gx = hist[t] @ Wih.T + bih
        gh = h @ Whh.T + bhh
        xr, xz, xn = jnp.split(gx, 3)
        hr, hz, hn = jnp.split(gh, 3)
        r = jax.nn.sigmoid(xr + hr)
        z = jax.nn.sigmoid(xz + hz)
        n = jnp.tanh(xn + r * hn)
        h = (1.0 - z) * n + z * h
        outs.append(h)
    return jnp.stack(outs, axis=0)


def kernel(x, edge_index, edge_attr, history, c1_Wl, c1_bl, c1_Wr, c1_br, c1_We, c1_att, c1_b, c2_Wl, c2_bl, c2_Wr, c2_br, c2_We, c2_att, c2_b, p_Win, p_bin, p_Wout, p_bout, m_W1, m_b1, m_W2, m_b2, g_Wih, g_Whh, g_bih, g_bhh, h_W1, h_b1, h_W2, h_b2):
    f32 = jnp.float32
    src = edge_index[0]
    dst = edge_index[1]

    colsum = _ea_colsum(edge_attr.reshape(E * EDIM // 128, 128))
    mean_ea = colsum[0].reshape(32, EDIM).sum(axis=0) * (1.0 / E)

    we1t = c1_We.T
    we2t = c2_We.T
    efm1 = (mean_ea @ we1t).reshape(1, HID)
    efm2 = (mean_ea @ we2t).reshape(1, HID)
    att1 = c1_att.reshape(1, HID)
    att2 = c2_att.reshape(1, HID)
    zeros_hbm = jnp.zeros((N, ROWW), f32)
    zeros1_hbm = jnp.zeros((N,), f32)
    one = jnp.ones((1, 1), f32)
    zero = jnp.zeros((1, 1), f32)

    pre1 = _pre(x, c1_Wl, c1_bl.reshape(1, HID), c1_Wr, c1_br.reshape(1, HID),
                efm1, att1)
    pre2 = _gat_layer(pre1, src, dst, edge_attr, we1t, att1,
                      c1_b.reshape(1, HID), zeros_hbm, zeros1_hbm,
                      c2_Wl, c2_bl.reshape(1, HID), c2_Wr,
                      c2_br.reshape(1, HID), efm2, att2, one)
    eye = jnp.eye(HID, dtype=f32)
    zb = jnp.zeros((1, HID), f32)
    h2 = _gat_layer(pre2, src, dst, edge_attr, we2t, att2,
                    c2_b.reshape(1, HID), zeros_hbm, zeros1_hbm,
                    eye, zb, eye, zb, zb, zb, zero)[0]

    a = jnp.max(_mha_max(h2, p_Win, p_bin, p_Wout, p_bout), axis=0)
    m = jax.nn.leaky_relu(a.reshape(1, -1) @ m_W1.T + m_b1, 0.01)
    m = jax.nn.leaky_relu(m @ m_W2.T + m_b2, 0.01)
    g = _gru(history, g_Wih, g_Whh, g_bih, g_bhh, HID).reshape(1, -1)
    cat = jnp.concatenate([m, g], axis=1)
    o = jax.nn.leaky_relu(cat @ h_W1.T + h_b1, 0.01)
    return o @ h_W2.T + h_b2
